# baseline (speedup 1.0000x reference)
import functools
import jax
import jax.numpy as jnp
from jax.experimental import pallas as pl
from jax.experimental.pallas import tpu as pltpu

NS = 16
SP = 8
STRIDES = [1, 4, 4, 4, 4]

_F32_NEG = float('-inf')
_F32_INF = float('inf')
_I32_BIG = 2 ** 30


def _pad_to(x, mult):
    return (x + mult - 1) // mult * mult


def _fps_body(px_ref, py_ref, pz_ref, idx_ref, dist_ref, *, m, n, orows):
    R = px_ref.shape[0]
    px = px_ref[...]
    py = py_ref[...]
    pz = pz_ref[...]
    row = jax.lax.broadcasted_iota(jnp.int32, (R, 128), 0)
    col = jax.lax.broadcasted_iota(jnp.int32, (R, 128), 1)
    flat = row * 128 + col
    valid = flat < n
    lx0 = px[0, 0]
    ly0 = py[0, 0]
    lz0 = pz[0, 0]
    d0 = (px - lx0) ** 2 + (py - ly0) ** 2 + (pz - lz0) ** 2
    dist_ref[...] = jnp.where(valid, d0, _F32_NEG)
    arow = jax.lax.broadcasted_iota(jnp.int32, (orows, 128), 0)
    acol = jax.lax.broadcasted_iota(jnp.int32, (orows, 128), 1)
    aflat = arow * 128 + acol
    acc0 = jnp.zeros((orows, 128), jnp.int32)

    lane1 = jax.lax.broadcasted_iota(jnp.int32, (1, 128), 1)

    def body(i, carry):
        lx, ly, lz, acc = carry
        d = (px - lx) ** 2 + (py - ly) ** 2 + (pz - lz) ** 2
        dist = jnp.minimum(dist_ref[...], d)
        dist_ref[...] = dist
        gmax = jnp.max(dist)
        nxt = jnp.min(jnp.where(dist == gmax, flat, _I32_BIG))
        r = nxt // 128
        sel1 = lane1 == (nxt % 128)
        nlx = jnp.max(jnp.where(sel1, px_ref[pl.ds(r, 1), :], _F32_NEG))
        nly = jnp.max(jnp.where(sel1, py_ref[pl.ds(r, 1), :], _F32_NEG))
        nlz = jnp.max(jnp.where(sel1, pz_ref[pl.ds(r, 1), :], _F32_NEG))
        acc = jnp.where(aflat == i, nxt, acc)
        return (nlx, nly, nlz, acc)

    out = jax.lax.fori_loop(1, m, body, (lx0, ly0, lz0, acc0))
    idx_ref[...] = out[3]


def _fps_call(p, m):
    n = p.shape[0]
    npad = _pad_to(n, 1024)
    R = npad // 128
    pp = jnp.pad(p, ((0, npad - n), (0, 0)))
    px = pp[:, 0].reshape(R, 128)
    py = pp[:, 1].reshape(R, 128)
    pz = pp[:, 2].reshape(R, 128)
    orows = _pad_to(_pad_to(m, 128) // 128, 8)
    out = pl.pallas_call(
        functools.partial(_fps_body, m=m, n=n, orows=orows),
        in_specs=[pl.BlockSpec(px.shape, lambda: (0, 0))] * 3,
        out_specs=pl.BlockSpec((orows, 128), lambda: (0, 0)),
        out_shape=jax.ShapeDtypeStruct((orows, 128), jnp.int32),
        scratch_shapes=[pltpu.VMEM((R, 128), jnp.float32)],
    )(px, py, pz)
    return out.reshape(-1)[:m]


def _knn_body(q_ref, rx_ref, ry_ref, rz_ref, oi_ref, od_ref, *, k, rb, nchunk):
    qb = q_ref.shape[0]
    qx = jax.lax.broadcast_in_dim(q_ref[:, 0:1], (qb, rb), (0, 1))
    qy = jax.lax.broadcast_in_dim(q_ref[:, 1:2], (qb, rb), (0, 1))
    qz = jax.lax.broadcast_in_dim(q_ref[:, 2:3], (qb, rb), (0, 1))
    lane = jax.lax.broadcasted_iota(jnp.int32, (qb, rb), 1)

    def chunk(c, carry):
        run_d, run_i = carry
        off = c * rb
        rx = jax.lax.broadcast_in_dim(
            rx_ref[pl.ds(0, 1), pl.ds(off, rb)], (qb, rb), (0, 1))
        ry = jax.lax.broadcast_in_dim(
            ry_ref[pl.ds(0, 1), pl.ds(off, rb)], (qb, rb), (0, 1))
        rz = jax.lax.broadcast_in_dim(
            rz_ref[pl.ds(0, 1), pl.ds(off, rb)], (qb, rb), (0, 1))
        d = (qx - rx) ** 2 + (qy - ry) ** 2 + (qz - rz) ** 2
        cand_d = jnp.concatenate([run_d, d], axis=1)
        cand_i = jnp.concatenate([run_i, off + lane], axis=1)
        nd, ni = [], []
        for _ in range(k):
            mval = jnp.min(cand_d, axis=1, keepdims=True)
            eqm = cand_d == mval
            ival = jnp.min(jnp.where(eqm, cand_i, _I32_BIG), axis=1,
                           keepdims=True)
            cand_d = jnp.where(eqm & (cand_i == ival), _F32_INF, cand_d)
            nd.append(mval)
            ni.append(ival)
        return jnp.concatenate(nd, axis=1), jnp.concatenate(ni, axis=1)

    init_d = jnp.full((qb, k), _F32_INF, jnp.float32)
    init_i = jnp.zeros((qb, k), jnp.int32)
    run_d, run_i = jax.lax.fori_loop(0, nchunk, chunk, (init_d, init_i))
    pad = jnp.zeros((qb, 128 - k), jnp.float32)
    oi_ref[...] = jnp.concatenate([run_i, pad.astype(jnp.int32)], axis=1)
    od_ref[...] = jnp.concatenate([run_d, pad], axis=1)


def _knn_sorted_body(q_ref, rx_ref, ry_ref, rz_ref, ri_ref, oi_ref, od_ref,
                     *, k, rb, nchunk, nqp, nrp):
    qb = q_ref.shape[0]
    qx = jax.lax.broadcast_in_dim(q_ref[:, 0:1], (qb, rb), (0, 1))
    qy = jax.lax.broadcast_in_dim(q_ref[:, 1:2], (qb, rb), (0, 1))
    qz = jax.lax.broadcast_in_dim(q_ref[:, 2:3], (qb, rb), (0, 1))
    qx1 = q_ref[:, 0:1]

    def merge(off, run_d, run_i):
        rx = jax.lax.broadcast_in_dim(
            rx_ref[pl.ds(0, 1), pl.ds(off, rb)], (qb, rb), (0, 1))
        ry = jax.lax.broadcast_in_dim(
            ry_ref[pl.ds(0, 1), pl.ds(off, rb)], (qb, rb), (0, 1))
        rz = jax.lax.broadcast_in_dim(
            rz_ref[pl.ds(0, 1), pl.ds(off, rb)], (qb, rb), (0, 1))
        rid = jax.lax.broadcast_in_dim(
            ri_ref[pl.ds(0, 1), pl.ds(off, rb)], (qb, rb), (0, 1))
        d = (qx - rx) ** 2 + (qy - ry) ** 2 + (qz - rz) ** 2
        cand_d = jnp.concatenate([run_d, d], axis=1)
        cand_i = jnp.concatenate([run_i, rid], axis=1)
        nd, ni = [], []
        for _ in range(k):
            mval = jnp.min(cand_d, axis=1, keepdims=True)
            eqm = cand_d == mval
            ival = jnp.min(jnp.where(eqm, cand_i, _I32_BIG), axis=1,
                           keepdims=True)
            cand_d = jnp.where(eqm & (cand_i == ival), _F32_INF, cand_d)
            nd.append(mval)
            ni.append(ival)
        return jnp.concatenate(nd, axis=1), jnp.concatenate(ni, axis=1)

    i = pl.program_id(0)
    c0 = jnp.minimum(((i * qb + qb // 2) * nrp) // (nqp * rb), nchunk - 1)
    init_d = jnp.full((qb, k), _F32_INF, jnp.float32)
    init_i = jnp.zeros((qb, k), jnp.int32)
    run_d, run_i = merge(c0 * rb, init_d, init_i)
    l0 = c0 - 1
    r0 = c0 + 1
    st0 = (l0, r0, l0 < 0, r0 >= nchunk, run_d, run_i)

    def wcond(st):
        return jnp.logical_not(jnp.logical_and(st[2], st[3]))

    def wbody(st):
        l, r, ld, rd, run_d, run_i = st

        def lactive(a):
            rdl, ril = a
            off = l * rb
            xm = jnp.max(rx_ref[pl.ds(0, 1), pl.ds(off, rb)])
            t = rdl[:, k - 1:k]
            stop = jnp.all((qx1 > xm) & ((qx1 - xm) ** 2 > t))

            def proc(b):
                d2, i2 = merge(off, b[0], b[1])
                return d2, i2, l - 1, l - 1 < 0

            def halt(b):
                return b[0], b[1], l, jnp.bool_(True)

            return jax.lax.cond(stop, halt, proc, (rdl, ril))

        def lskip(a):
            return a[0], a[1], l, jnp.bool_(True)

        run_d, run_i, l, ld = jax.lax.cond(ld, lskip, lactive, (run_d, run_i))

        def ractive(a):
            rdr, rir = a
            off = r * rb
            xn = jnp.min(rx_ref[pl.ds(0, 1), pl.ds(off, rb)])
            t = rdr[:, k - 1:k]
            stop = jnp.all((qx1 < xn) & ((xn - qx1) ** 2 > t))

            def proc(b):
                d2, i2 = merge(off, b[0], b[1])
                return d2, i2, r + 1, r + 1 >= nchunk

            def halt(b):
                return b[0], b[1], r, jnp.bool_(True)

            return jax.lax.cond(stop, halt, proc, (rdr, rir))

        def rskip(a):
            return a[0], a[1], r, jnp.bool_(True)

        run_d, run_i, r, rd = jax.lax.cond(rd, rskip, ractive, (run_d, run_i))
        return (l, r, ld, rd, run_d, run_i)

    st = jax.lax.while_loop(wcond, wbody, st0)
    run_d, run_i = st[4], st[5]
    pad = jnp.zeros((qb, 128 - k), jnp.float32)
    oi_ref[...] = jnp.concatenate([run_i, pad.astype(jnp.int32)], axis=1)
    od_ref[...] = jnp.concatenate([run_d, pad], axis=1)


def _knn_sorted(q, r, k, rb, nrp, nchunk):
    nq, nr = q.shape[0], r.shape[0]
    if q is r:
        rperm = jnp.argsort(r[:, 0])
        rs = r[rperm]
        qperm, qs = rperm, rs
    else:
        rperm = jnp.argsort(r[:, 0])
        rs = r[rperm]
        qperm = jnp.argsort(q[:, 0])
        qs = q[qperm]
    qblk = 256
    nqp = _pad_to(nq, qblk)
    qp = jnp.pad(qs, ((0, nqp - nq), (0, 0)), mode='edge')
    q3 = jnp.pad(qp, ((0, 0), (0, 125)))
    rpad = jnp.pad(rs, ((0, nrp - nr), (0, 0)),
                   constant_values=jnp.float32(1e18))
    r8 = jnp.broadcast_to(rpad.T[:, None, :], (3, 8, nrp))
    rx, ry, rz = r8[0], r8[1], r8[2]
    ripad = jnp.pad(rperm.astype(jnp.int32), (0, nrp - nr))
    ri8 = jnp.broadcast_to(ripad[None, :], (8, nrp))
    grid = nqp // qblk
    oi, od = pl.pallas_call(
        functools.partial(_knn_sorted_body, k=k, rb=rb, nchunk=nchunk,
                          nqp=nqp, nrp=nrp),
        grid=(grid,),
        in_specs=[
            pl.BlockSpec((qblk, 128), lambda i: (i, 0)),
            pl.BlockSpec((8, nrp), lambda i: (0, 0)),
            pl.BlockSpec((8, nrp), lambda i: (0, 0)),
            pl.BlockSpec((8, nrp), lambda i: (0, 0)),
            pl.BlockSpec((8, nrp), lambda i: (0, 0)),
        ],
        out_specs=[
            pl.BlockSpec((qblk, 128), lambda i: (i, 0)),
            pl.BlockSpec((qblk, 128), lambda i: (i, 0)),
        ],
        out_shape=[
            jax.ShapeDtypeStruct((nqp, 128), jnp.int32),
            jax.ShapeDtypeStruct((nqp, 128), jnp.float32),
        ],
    )(q3, rx, ry, rz, ri8)
    idx_s = oi[:nq, :k]
    dist_s = jnp.sqrt(jnp.maximum(od[:nq, :k], 0.0))
    idx = jnp.zeros_like(idx_s).at[qperm].set(idx_s)
    dist = jnp.zeros_like(dist_s).at[qperm].set(dist_s)
    return idx, dist


def _knn_call(q, r, k):
    nq, nr = q.shape[0], r.shape[0]
    rb = 512
    nrp = _pad_to(nr, rb)
    if nrp // rb >= 4 and nq >= 256:
        return _knn_sorted(q, r, k, rb, nrp, nrp // rb)
    qblk = 256 if nq >= 256 else _pad_to(nq, 8)
    nqp = _pad_to(nq, qblk)
    qp = jnp.pad(q, ((0, nqp - nq), (0, 0)))
    q3 = jnp.pad(qp, ((0, 0), (0, 125)))
    rpad = jnp.pad(r, ((0, nrp - nr), (0, 0)),
                   constant_values=jnp.float32(1e18))
    r8 = jnp.broadcast_to(rpad.T[:, None, :], (3, 8, nrp))
    rx, ry, rz = r8[0], r8[1], r8[2]
    grid = nqp // qblk
    oi, od = pl.pallas_call(
        functools.partial(_knn_body, k=k, rb=rb, nchunk=nrp // rb),
        grid=(grid,),
        in_specs=[
            pl.BlockSpec((qblk, 128), lambda i: (i, 0)),
            pl.BlockSpec((8, nrp), lambda i: (0, 0)),
            pl.BlockSpec((8, nrp), lambda i: (0, 0)),
            pl.BlockSpec((8, nrp), lambda i: (0, 0)),
        ],
        out_specs=[
            pl.BlockSpec((qblk, 128), lambda i: (i, 0)),
            pl.BlockSpec((qblk, 128), lambda i: (i, 0)),
        ],
        out_shape=[
            jax.ShapeDtypeStruct((nqp, 128), jnp.int32),
            jax.ShapeDtypeStruct((nqp, 128), jnp.float32),
        ],
    )(q3, rx, ry, rz)
    idx = oi[:nq, :k]
    dist = jnp.sqrt(jnp.maximum(od[:nq, :k], 0.0))
    return idx, dist


def _pt_layer(pr, p, x, idx):
    q = x @ pr['wq'] + pr['bq']
    kf = x @ pr['wk'] + pr['bk']
    v = x @ pr['wv'] + pr['bv']
    n, c = kf.shape
    ns = idx.shape[1]
    cs = c // SP
    eye = jnp.eye(ns, dtype=jnp.float32)
    g = jnp.concatenate([p, kf, v], axis=1)[idx]
    pr2 = (g[:, :, :3] - p[:, None, :]).reshape(n, ns * 3)
    xk2 = g[:, :, 3:3 + c].reshape(n, ns * c)
    xv2 = g[:, :, 3 + c:].reshape(n, ns * c)
    pe2 = jax.nn.relu(pr2 @ jnp.kron(eye, pr['wp1']) + jnp.tile(pr['bp1'], ns))
    pe2 = pe2 @ jnp.kron(eye, pr['wp2']) + jnp.tile(pr['bp2'], ns)
    w = jax.nn.relu(xk2 - jnp.tile(q, (1, ns)) + pe2)
    w = jax.nn.relu(w @ jnp.kron(eye, pr['ww1']) + jnp.tile(pr['bw1'], ns))
    w = w @ jnp.kron(eye, pr['ww2']) + jnp.tile(pr['bw2'], ns)
    w = jax.nn.softmax(w.reshape(n, ns, cs), axis=1)
    out = ((xv2 + pe2).reshape(n, ns, SP, cs) * w[:, :, None, :]).sum(1).reshape(n, c)
    return out


def _pt_block(bp, p, x, idx):
    identity = x
    h = jax.nn.relu(x @ bp['w1'])
    h = jax.nn.relu(_pt_layer(bp['layer'], p, h, idx))
    h = h @ bp['w3']
    return jax.nn.relu(h + identity)


def _cls_body(x_ref, w1_ref, b1_ref, w2_ref, b2_ref, o_ref):
    h = jax.nn.relu(x_ref[...] @ w1_ref[...] + b1_ref[...])
    o_ref[...] = h @ w2_ref[...] + b2_ref[...]


def _cls_head(x, w1, b1, w2, b2):
    n, c = x.shape
    nc = w2.shape[1]
    ncp = 128
    w2p = jnp.pad(w2, ((0, 0), (0, ncp - nc)))
    b2p = jnp.pad(b2, ((0, ncp - nc)))
    blk = 2000
    grid = (n + blk - 1) // blk
    out = pl.pallas_call(
        _cls_body,
        grid=(grid,),
        in_specs=[
            pl.BlockSpec((blk, c), lambda i: (i, 0)),
            pl.BlockSpec((c, c), lambda i: (0, 0)),
            pl.BlockSpec((c,), lambda i: (0,)),
            pl.BlockSpec((c, ncp), lambda i: (0, 0)),
            pl.BlockSpec((ncp,), lambda i: (0,)),
        ],
        out_specs=pl.BlockSpec((blk, ncp), lambda i: (i, 0)),
        out_shape=jax.ShapeDtypeStruct((n, ncp), jnp.float32),
    )(x, w1, b1, w2p, b2p)
    return out[:, :nc]


def _forward(params, p0, x0):
    xc = jnp.concatenate([p0, x0], axis=1)
    p = p0
    ps, xs, idxs = [], [], []
    for i in range(5):
        ep = params['enc'][i]
        if STRIDES[i] == 1:
            xc = jax.nn.relu(xc @ ep['down_w'])
        else:
            m = p.shape[0] // STRIDES[i]
            sidx = _fps_call(p, m)
            npts = p[sidx]
            gidx, _ = _knn_call(npts, p, NS)
            gg = jnp.concatenate([p, xc], axis=1)[gidx]
            grouped = jnp.concatenate(
                [gg[:, :, :3] - npts[:, None, :], gg[:, :, 3:]], -1)
            xc = jax.nn.relu(grouped @ ep['down_w']).max(axis=1)
            p = npts
        nidx, _ = _knn_call(p, p, NS)
        for bp in ep['blocks']:
            xc = _pt_block(bp, p, xc, nidx)
        ps.append(p); xs.append(xc); idxs.append(nidx)
    dp = params['dec'][4]
    g = jax.nn.relu(jnp.mean(xs[4], axis=0, keepdims=True) @ dp['w2'] + dp['b2'])
    h = jnp.concatenate([xs[4], jnp.broadcast_to(g, (xs[4].shape[0], g.shape[1]))], axis=1)
    xc = jax.nn.relu(h @ dp['w1'] + dp['b1'])
    xc = _pt_block(dp['blocks'][0], ps[4], xc, idxs[4])
    xs[4] = xc
    for i in (3, 2, 1, 0):
        dp = params['dec'][i]
        y1 = jax.nn.relu(xs[i] @ dp['w1'] + dp['b1'])
        y2 = jax.nn.relu(xs[i + 1] @ dp['w2'] + dp['b2'])
        ki, kd = _knn_call(ps[i], ps[i + 1], 3)
        wgt = 1.0 / (kd + 1e-8)
        wgt = wgt / wgt.sum(axis=1, keepdims=True)
        xc = y1 + (y2[ki] * wgt[:, :, None]).sum(axis=1)
        xc = _pt_block(dp['blocks'][0], ps[i], xc, idxs[i])
        xs[i] = xc
    return _cls_head(xs[0], params['cls_w1'], params['cls_b1'],
                     params['cls_w2'], params['cls_b2'])


def kernel(p0, x0, o0, params):
    return _forward(params, p0, x0)


# speedup vs baseline: 1.0690x; 1.0690x over previous
import functools
import jax
import jax.numpy as jnp
from jax.experimental import pallas as pl
from jax.experimental.pallas import tpu as pltpu

NS = 16
SP = 8
STRIDES = [1, 4, 4, 4, 4]

_F32_NEG = float('-inf')
_F32_INF = float('inf')
_I32_BIG = 2 ** 30


def _pad_to(x, mult):
    return (x + mult - 1) // mult * mult


def _fps_body(px_ref, py_ref, pz_ref, idx_ref, dist_ref, *, m, n, orows):
    R = px_ref.shape[0]
    px = px_ref[...]
    py = py_ref[...]
    pz = pz_ref[...]
    row = jax.lax.broadcasted_iota(jnp.int32, (R, 128), 0)
    col = jax.lax.broadcasted_iota(jnp.int32, (R, 128), 1)
    flat = row * 128 + col
    valid = flat < n
    lx0 = px[0, 0]
    ly0 = py[0, 0]
    lz0 = pz[0, 0]
    d0 = (px - lx0) ** 2 + (py - ly0) ** 2 + (pz - lz0) ** 2
    dist_ref[...] = jnp.where(valid, d0, _F32_NEG)
    arow = jax.lax.broadcasted_iota(jnp.int32, (orows, 128), 0)
    acol = jax.lax.broadcasted_iota(jnp.int32, (orows, 128), 1)
    aflat = arow * 128 + acol
    acc0 = jnp.zeros((orows, 128), jnp.int32)

    lane1 = jax.lax.broadcasted_iota(jnp.int32, (1, 128), 1)

    def body(i, carry):
        lx, ly, lz, acc = carry
        d = (px - lx) ** 2 + (py - ly) ** 2 + (pz - lz) ** 2
        dist = jnp.minimum(dist_ref[...], d)
        dist_ref[...] = dist
        gmax = jnp.max(dist)
        nxt = jnp.min(jnp.where(dist == gmax, flat, _I32_BIG))
        r = nxt // 128
        sel1 = lane1 == (nxt % 128)
        nlx = jnp.max(jnp.where(sel1, px_ref[pl.ds(r, 1), :], _F32_NEG))
        nly = jnp.max(jnp.where(sel1, py_ref[pl.ds(r, 1), :], _F32_NEG))
        nlz = jnp.max(jnp.where(sel1, pz_ref[pl.ds(r, 1), :], _F32_NEG))
        acc = jnp.where(aflat == i, nxt, acc)
        return (nlx, nly, nlz, acc)

    out = jax.lax.fori_loop(1, m, body, (lx0, ly0, lz0, acc0))
    idx_ref[...] = out[3]


def _fps_call(p, m):
    n = p.shape[0]
    npad = _pad_to(n, 1024)
    R = npad // 128
    pp = jnp.pad(p, ((0, npad - n), (0, 0)))
    px = pp[:, 0].reshape(R, 128)
    py = pp[:, 1].reshape(R, 128)
    pz = pp[:, 2].reshape(R, 128)
    orows = _pad_to(_pad_to(m, 128) // 128, 8)
    out = pl.pallas_call(
        functools.partial(_fps_body, m=m, n=n, orows=orows),
        in_specs=[pl.BlockSpec(px.shape, lambda: (0, 0))] * 3,
        out_specs=pl.BlockSpec((orows, 128), lambda: (0, 0)),
        out_shape=jax.ShapeDtypeStruct((orows, 128), jnp.int32),
        scratch_shapes=[pltpu.VMEM((R, 128), jnp.float32)],
    )(px, py, pz)
    return out.reshape(-1)[:m]


def _knn_body(q_ref, rx_ref, ry_ref, rz_ref, oi_ref, od_ref, *, k, rb, nchunk):
    qb = q_ref.shape[0]
    qx = jax.lax.broadcast_in_dim(q_ref[:, 0:1], (qb, rb), (0, 1))
    qy = jax.lax.broadcast_in_dim(q_ref[:, 1:2], (qb, rb), (0, 1))
    qz = jax.lax.broadcast_in_dim(q_ref[:, 2:3], (qb, rb), (0, 1))
    lane = jax.lax.broadcasted_iota(jnp.int32, (qb, rb), 1)

    def chunk(c, carry):
        run_d, run_i = carry
        off = c * rb
        rx = jax.lax.broadcast_in_dim(
            rx_ref[pl.ds(0, 1), pl.ds(off, rb)], (qb, rb), (0, 1))
        ry = jax.lax.broadcast_in_dim(
            ry_ref[pl.ds(0, 1), pl.ds(off, rb)], (qb, rb), (0, 1))
        rz = jax.lax.broadcast_in_dim(
            rz_ref[pl.ds(0, 1), pl.ds(off, rb)], (qb, rb), (0, 1))
        d = (qx - rx) ** 2 + (qy - ry) ** 2 + (qz - rz) ** 2
        cand_d = jnp.concatenate([run_d, d], axis=1)
        cand_i = jnp.concatenate([run_i, off + lane], axis=1)
        nd, ni = [], []
        for _ in range(k):
            mval = jnp.min(cand_d, axis=1, keepdims=True)
            eqm = cand_d == mval
            ival = jnp.min(jnp.where(eqm, cand_i, _I32_BIG), axis=1,
                           keepdims=True)
            cand_d = jnp.where(eqm & (cand_i == ival), _F32_INF, cand_d)
            nd.append(mval)
            ni.append(ival)
        return jnp.concatenate(nd, axis=1), jnp.concatenate(ni, axis=1)

    init_d = jnp.full((qb, k), _F32_INF, jnp.float32)
    init_i = jnp.zeros((qb, k), jnp.int32)
    run_d, run_i = jax.lax.fori_loop(0, nchunk, chunk, (init_d, init_i))
    pad = jnp.zeros((qb, 128 - k), jnp.float32)
    oi_ref[...] = jnp.concatenate([run_i, pad.astype(jnp.int32)], axis=1)
    od_ref[...] = jnp.concatenate([run_d, pad], axis=1)


def _knn_sorted_body(q_ref, rx_ref, ry_ref, rz_ref, ri_ref, oi_ref, od_ref,
                     *, k, rb, nchunk, nqp, nrp):
    qb = q_ref.shape[0]
    qx = jax.lax.broadcast_in_dim(q_ref[:, 0:1], (qb, rb), (0, 1))
    qy = jax.lax.broadcast_in_dim(q_ref[:, 1:2], (qb, rb), (0, 1))
    qz = jax.lax.broadcast_in_dim(q_ref[:, 2:3], (qb, rb), (0, 1))
    qx1 = q_ref[:, 0:1]

    def merge(off, run_d, run_i):
        rx = jax.lax.broadcast_in_dim(
            rx_ref[pl.ds(0, 1), pl.ds(off, rb)], (qb, rb), (0, 1))
        ry = jax.lax.broadcast_in_dim(
            ry_ref[pl.ds(0, 1), pl.ds(off, rb)], (qb, rb), (0, 1))
        rz = jax.lax.broadcast_in_dim(
            rz_ref[pl.ds(0, 1), pl.ds(off, rb)], (qb, rb), (0, 1))
        rid = jax.lax.broadcast_in_dim(
            ri_ref[pl.ds(0, 1), pl.ds(off, rb)], (qb, rb), (0, 1))
        d = (qx - rx) ** 2 + (qy - ry) ** 2 + (qz - rz) ** 2
        cand_d = jnp.concatenate([run_d, d], axis=1)
        cand_i = jnp.concatenate([run_i, rid], axis=1)
        nd, ni = [], []
        for _ in range(k):
            mval = jnp.min(cand_d, axis=1, keepdims=True)
            eqm = cand_d == mval
            ival = jnp.min(jnp.where(eqm, cand_i, _I32_BIG), axis=1,
                           keepdims=True)
            cand_d = jnp.where(eqm & (cand_i == ival), _F32_INF, cand_d)
            nd.append(mval)
            ni.append(ival)
        return jnp.concatenate(nd, axis=1), jnp.concatenate(ni, axis=1)

    i = pl.program_id(0)
    c0 = jnp.minimum(((i * qb + qb // 2) * nrp) // (nqp * rb), nchunk - 1)
    init_d = jnp.full((qb, k), _F32_INF, jnp.float32)
    init_i = jnp.zeros((qb, k), jnp.int32)
    run_d, run_i = merge(c0 * rb, init_d, init_i)
    l0 = c0 - 1
    r0 = c0 + 1
    st0 = (l0, r0, l0 < 0, r0 >= nchunk, run_d, run_i)

    def wcond(st):
        return jnp.logical_not(jnp.logical_and(st[2], st[3]))

    def wbody(st):
        l, r, ld, rd, run_d, run_i = st

        def lactive(a):
            rdl, ril = a
            off = l * rb
            xm = jnp.max(rx_ref[pl.ds(0, 1), pl.ds(off, rb)])
            t = rdl[:, k - 1:k]
            stop = jnp.all((qx1 > xm) & ((qx1 - xm) ** 2 > t))

            def proc(b):
                d2, i2 = merge(off, b[0], b[1])
                return d2, i2, l - 1, l - 1 < 0

            def halt(b):
                return b[0], b[1], l, jnp.bool_(True)

            return jax.lax.cond(stop, halt, proc, (rdl, ril))

        def lskip(a):
            return a[0], a[1], l, jnp.bool_(True)

        run_d, run_i, l, ld = jax.lax.cond(ld, lskip, lactive, (run_d, run_i))

        def ractive(a):
            rdr, rir = a
            off = r * rb
            xn = jnp.min(rx_ref[pl.ds(0, 1), pl.ds(off, rb)])
            t = rdr[:, k - 1:k]
            stop = jnp.all((qx1 < xn) & ((xn - qx1) ** 2 > t))

            def proc(b):
                d2, i2 = merge(off, b[0], b[1])
                return d2, i2, r + 1, r + 1 >= nchunk

            def halt(b):
                return b[0], b[1], r, jnp.bool_(True)

            return jax.lax.cond(stop, halt, proc, (rdr, rir))

        def rskip(a):
            return a[0], a[1], r, jnp.bool_(True)

        run_d, run_i, r, rd = jax.lax.cond(rd, rskip, ractive, (run_d, run_i))
        return (l, r, ld, rd, run_d, run_i)

    st = jax.lax.while_loop(wcond, wbody, st0)
    run_d, run_i = st[4], st[5]
    pad = jnp.zeros((qb, 128 - k), jnp.float32)
    oi_ref[...] = jnp.concatenate([run_i, pad.astype(jnp.int32)], axis=1)
    od_ref[...] = jnp.concatenate([run_d, pad], axis=1)


def _knn_sorted(q, r, k, rb, nrp, nchunk):
    nq, nr = q.shape[0], r.shape[0]
    if q is r:
        rperm = jnp.argsort(r[:, 0])
        rs = r[rperm]
        qperm, qs = rperm, rs
    else:
        rperm = jnp.argsort(r[:, 0])
        rs = r[rperm]
        qperm = jnp.argsort(q[:, 0])
        qs = q[qperm]
    qblk = 256
    nqp = _pad_to(nq, qblk)
    qp = jnp.pad(qs, ((0, nqp - nq), (0, 0)), mode='edge')
    q3 = jnp.pad(qp, ((0, 0), (0, 125)))
    rpad = jnp.pad(rs, ((0, nrp - nr), (0, 0)),
                   constant_values=jnp.float32(1e18))
    r8 = jnp.broadcast_to(rpad.T[:, None, :], (3, 8, nrp))
    rx, ry, rz = r8[0], r8[1], r8[2]
    ripad = jnp.pad(rperm.astype(jnp.int32), (0, nrp - nr))
    ri8 = jnp.broadcast_to(ripad[None, :], (8, nrp))
    grid = nqp // qblk
    oi, od = pl.pallas_call(
        functools.partial(_knn_sorted_body, k=k, rb=rb, nchunk=nchunk,
                          nqp=nqp, nrp=nrp),
        grid=(grid,),
        in_specs=[
            pl.BlockSpec((qblk, 128), lambda i: (i, 0)),
            pl.BlockSpec((8, nrp), lambda i: (0, 0)),
            pl.BlockSpec((8, nrp), lambda i: (0, 0)),
            pl.BlockSpec((8, nrp), lambda i: (0, 0)),
            pl.BlockSpec((8, nrp), lambda i: (0, 0)),
        ],
        out_specs=[
            pl.BlockSpec((qblk, 128), lambda i: (i, 0)),
            pl.BlockSpec((qblk, 128), lambda i: (i, 0)),
        ],
        out_shape=[
            jax.ShapeDtypeStruct((nqp, 128), jnp.int32),
            jax.ShapeDtypeStruct((nqp, 128), jnp.float32),
        ],
    )(q3, rx, ry, rz, ri8)
    idx_s = oi[:nq, :k]
    dist_s = jnp.sqrt(jnp.maximum(od[:nq, :k], 0.0))
    idx = jnp.zeros_like(idx_s).at[qperm].set(idx_s)
    dist = jnp.zeros_like(dist_s).at[qperm].set(dist_s)
    return idx, dist


def _knn_call(q, r, k):
    nq, nr = q.shape[0], r.shape[0]
    rb = 512
    nrp = _pad_to(nr, rb)
    if nrp // rb >= 4 and nq >= 256:
        return _knn_sorted(q, r, k, rb, nrp, nrp // rb)
    qblk = 256 if nq >= 256 else _pad_to(nq, 8)
    nqp = _pad_to(nq, qblk)
    qp = jnp.pad(q, ((0, nqp - nq), (0, 0)))
    q3 = jnp.pad(qp, ((0, 0), (0, 125)))
    rpad = jnp.pad(r, ((0, nrp - nr), (0, 0)),
                   constant_values=jnp.float32(1e18))
    r8 = jnp.broadcast_to(rpad.T[:, None, :], (3, 8, nrp))
    rx, ry, rz = r8[0], r8[1], r8[2]
    grid = nqp // qblk
    oi, od = pl.pallas_call(
        functools.partial(_knn_body, k=k, rb=rb, nchunk=nrp // rb),
        grid=(grid,),
        in_specs=[
            pl.BlockSpec((qblk, 128), lambda i: (i, 0)),
            pl.BlockSpec((8, nrp), lambda i: (0, 0)),
            pl.BlockSpec((8, nrp), lambda i: (0, 0)),
            pl.BlockSpec((8, nrp), lambda i: (0, 0)),
        ],
        out_specs=[
            pl.BlockSpec((qblk, 128), lambda i: (i, 0)),
            pl.BlockSpec((qblk, 128), lambda i: (i, 0)),
        ],
        out_shape=[
            jax.ShapeDtypeStruct((nqp, 128), jnp.int32),
            jax.ShapeDtypeStruct((nqp, 128), jnp.float32),
        ],
    )(q3, rx, ry, rz)
    idx = oi[:nq, :k]
    dist = jnp.sqrt(jnp.maximum(od[:nq, :k], 0.0))
    return idx, dist


def _pt_layer(pr, p, x, idx):
    q = x @ pr['wq'] + pr['bq']
    kf = x @ pr['wk'] + pr['bk']
    v = x @ pr['wv'] + pr['bv']
    c = kf.shape[1]
    g = jnp.concatenate([p, kf, v], axis=1)[idx]
    p_r = g[:, :, :3] - p[:, None, :]
    xk = g[:, :, 3:3 + c]
    xv = g[:, :, 3 + c:]
    pe = jax.nn.relu(p_r @ pr['wp1'] + pr['bp1'])
    pe = pe @ pr['wp2'] + pr['bp2']
    w = xk - q[:, None, :] + pe
    w = jax.nn.relu(w)
    w = jax.nn.relu(w @ pr['ww1'] + pr['bw1'])
    w = w @ pr['ww2'] + pr['bw2']
    w = jax.nn.softmax(w, axis=1)
    n, ns, c = xv.shape
    out = ((xv + pe).reshape(n, ns, SP, c // SP) * w[:, :, None, :]).sum(1).reshape(n, c)
    return out


def _pt_block(bp, p, x, idx):
    identity = x
    h = jax.nn.relu(x @ bp['w1'])
    h = jax.nn.relu(_pt_layer(bp['layer'], p, h, idx))
    h = h @ bp['w3']
    return jax.nn.relu(h + identity)


def _cls_body(x_ref, w1_ref, b1_ref, w2_ref, b2_ref, o_ref):
    h = jax.nn.relu(x_ref[...] @ w1_ref[...] + b1_ref[...])
    o_ref[...] = h @ w2_ref[...] + b2_ref[...]


def _cls_head(x, w1, b1, w2, b2):
    n, c = x.shape
    nc = w2.shape[1]
    ncp = 128
    w2p = jnp.pad(w2, ((0, 0), (0, ncp - nc)))
    b2p = jnp.pad(b2, ((0, ncp - nc)))
    blk = 2000
    grid = (n + blk - 1) // blk
    out = pl.pallas_call(
        _cls_body,
        grid=(grid,),
        in_specs=[
            pl.BlockSpec((blk, c), lambda i: (i, 0)),
            pl.BlockSpec((c, c), lambda i: (0, 0)),
            pl.BlockSpec((c,), lambda i: (0,)),
            pl.BlockSpec((c, ncp), lambda i: (0, 0)),
            pl.BlockSpec((ncp,), lambda i: (0,)),
        ],
        out_specs=pl.BlockSpec((blk, ncp), lambda i: (i, 0)),
        out_shape=jax.ShapeDtypeStruct((n, ncp), jnp.float32),
    )(x, w1, b1, w2p, b2p)
    return out[:, :nc]


def _forward(params, p0, x0):
    xc = jnp.concatenate([p0, x0], axis=1)
    p = p0
    ps, xs, idxs = [], [], []
    for i in range(5):
        ep = params['enc'][i]
        if STRIDES[i] == 1:
            xc = jax.nn.relu(xc @ ep['down_w'])
        else:
            m = p.shape[0] // STRIDES[i]
            sidx = _fps_call(p, m)
            npts = p[sidx]
            gidx, _ = _knn_call(npts, p, NS)
            gg = jnp.concatenate([p, xc], axis=1)[gidx]
            grouped = jnp.concatenate(
                [gg[:, :, :3] - npts[:, None, :], gg[:, :, 3:]], -1)
            xc = jax.nn.relu(grouped @ ep['down_w']).max(axis=1)
            p = npts
        nidx, _ = _knn_call(p, p, NS)
        for bp in ep['blocks']:
            xc = _pt_block(bp, p, xc, nidx)
        ps.append(p); xs.append(xc); idxs.append(nidx)
    dp = params['dec'][4]
    g = jax.nn.relu(jnp.mean(xs[4], axis=0, keepdims=True) @ dp['w2'] + dp['b2'])
    h = jnp.concatenate([xs[4], jnp.broadcast_to(g, (xs[4].shape[0], g.shape[1]))], axis=1)
    xc = jax.nn.relu(h @ dp['w1'] + dp['b1'])
    xc = _pt_block(dp['blocks'][0], ps[4], xc, idxs[4])
    xs[4] = xc
    for i in (3, 2, 1, 0):
        dp = params['dec'][i]
        y1 = jax.nn.relu(xs[i] @ dp['w1'] + dp['b1'])
        y2 = jax.nn.relu(xs[i + 1] @ dp['w2'] + dp['b2'])
        ki, kd = _knn_call(ps[i], ps[i + 1], 3)
        wgt = 1.0 / (kd + 1e-8)
        wgt = wgt / wgt.sum(axis=1, keepdims=True)
        xc = y1 + (y2[ki] * wgt[:, :, None]).sum(axis=1)
        xc = _pt_block(dp['blocks'][0], ps[i], xc, idxs[i])
        xs[i] = xc
    return _cls_head(xs[0], params['cls_w1'], params['cls_b1'],
                     params['cls_w2'], params['cls_b2'])


def kernel(p0, x0, o0, params):
    return _forward(params, p0, x0)
